# baseline (speedup 1.0000x reference)
import functools
import numpy as np
import jax
import jax.numpy as jnp
from jax import lax
from jax.experimental import pallas as pl
from jax.experimental.pallas import tpu as pltpu
from jax.experimental.pallas import tpu_sc as plsc

F = 26
B = 4096
D = 64
PER_TABLE = 38462
TOTAL_VOCAB = F * PER_TABLE
NUM_F = F + 1
N = F * B
NW = 32
ROWS_PER_W = N // NW
CHUNK = 128
NCHUNK = ROWS_PER_W // CHUNK
IDX_ROWS = N // CHUNK
BS = 128
_LI, _LJ = np.tril_indices(NUM_F, k=-1)


TW = 16384
HALF = 507904
TGRID = HALF // TW
PV_ROWS = HALF
_NBLK = -(-TOTAL_VOCAB // TW) - 1


def _tc_trans_body(tina, tinb, tout):
    tout[...] = jnp.concatenate(
        [jnp.transpose(tina[...]), jnp.transpose(tinb[...])], axis=1)


def _tc_transpose(tbl_t):
    return pl.pallas_call(
        _tc_trans_body,
        grid=(TGRID,),
        in_specs=[
            pl.BlockSpec((D, TW), lambda i: (0, i)),
            pl.BlockSpec((D, TW), lambda i: (0, jnp.minimum(TGRID + i, _NBLK))),
        ],
        out_specs=pl.BlockSpec((TW, 2 * D), lambda i: (i, 0)),
        out_shape=jax.ShapeDtypeStruct((PV_ROWS, 2 * D), jnp.float32),
    )(tbl_t, tbl_t)


@functools.lru_cache(maxsize=1)
def _make_sc_gather():
    mesh = plsc.VectorSubcoreMesh(core_axis_name="c", subcore_axis_name="s")

    @functools.partial(
        pl.kernel,
        mesh=mesh,
        out_type=jax.ShapeDtypeStruct((F, B, 2 * D), jnp.float32),
        scratch_types=[
            pltpu.VMEM((NCHUNK, CHUNK), jnp.int32),
            pltpu.VMEM((CHUNK, 2 * D), jnp.float32),
            pltpu.SemaphoreType.DMA,
        ],
    )
    def _sc_gather(idx_hbm, table_hbm, out_hbm, idx_v, rows_v, sem):
        wid = lax.axis_index("s") * 2 + lax.axis_index("c")
        pltpu.sync_copy(idx_hbm.at[wid], idx_v)
        for g in range(NCHUNK):
            pltpu.async_copy(table_hbm.at[idx_v.at[g]], rows_v, sem).wait()
            gidx = wid * NCHUNK + g
            f = gidx // (B // CHUNK)
            col = (gidx % (B // CHUNK)) * CHUNK
            pltpu.sync_copy(rows_v, out_hbm.at[f, pl.ds(col, CHUNK)])

    return _sc_gather


def _tc_body(df, sp, par, dw0, db0, dw1, db1, dw2, db2,
             wd, w3, ob0, ow1, ob1, ow2, ob2, ow3, ob3, out):
    f32 = jnp.float32
    x = jnp.maximum(jnp.dot(df[...], dw0[...], preferred_element_type=f32) + db0[...], 0.0)
    x = jnp.maximum(jnp.dot(x, dw1[...], preferred_element_type=f32) + db1[...], 0.0)
    dense_emb = jnp.maximum(jnp.dot(x, dw2[...], preferred_element_type=f32) + db2[...], 0.0)

    pairs = sp[...]
    csp = jnp.where(par[...] != 0, pairs[:, :, D:], pairs[:, :, :D])
    c = jnp.concatenate([dense_emb[None], csp], axis=0)
    cb = c.astype(jnp.bfloat16)
    z = lax.dot_general(cb, cb, (((2,), (2,)), ((1,), (1,))),
                        preferred_element_type=f32)

    bf16 = jnp.bfloat16
    y = jnp.dot(dense_emb, wd[...], preferred_element_type=f32) + ob0[...]
    zb = z.astype(bf16)
    for f in range(NUM_F):
        y = y + jnp.dot(zb[:, f, :], w3[f].astype(bf16), preferred_element_type=f32)
    y = jnp.maximum(y, 0.0)
    y = jnp.maximum(jnp.dot(y.astype(bf16), ow1[...].astype(bf16),
                            preferred_element_type=f32) + ob1[...], 0.0)
    y = jnp.maximum(jnp.dot(y.astype(bf16), ow2[...].astype(bf16),
                            preferred_element_type=f32) + ob2[...], 0.0)
    out[...] = jnp.dot(y, ow3[...], preferred_element_type=f32) + ob3[...]


def kernel(dense_features, sparse_values, sparse_offsets, emb_table,
           dense_w0, dense_b0, dense_w1, dense_b1, dense_w2, dense_b2,
           over_w0, over_b0, over_w1, over_b1, over_w2, over_b2,
           over_w3, over_b3):
    del sparse_offsets
    half_flag = sparse_values >= HALF
    pair_idx = jnp.where(half_flag, sparse_values - HALF,
                         sparse_values).reshape(NW, NCHUNK, CHUNK)
    parity = jnp.broadcast_to(
        half_flag.astype(jnp.int8).reshape(F, B)[:, :, None], (F, B, D))
    pair_view = _tc_transpose(emb_table.T)
    sp = _make_sc_gather()(pair_idx, pair_view)

    wd = over_w0[:D]
    w3 = jnp.zeros((NUM_F, NUM_F, over_w0.shape[1]), jnp.float32)
    w3 = w3.at[_LI, _LJ, :].set(over_w0[D:])

    grid = B // BS
    full = lambda a: pl.BlockSpec(a.shape, lambda i: (0,) * a.ndim)
    b2 = lambda b: b.reshape(1, -1)

    out = pl.pallas_call(
        _tc_body,
        grid=(grid,),
        in_specs=[
            pl.BlockSpec((BS, 13), lambda i: (i, 0)),
            pl.BlockSpec((F, BS, 2 * D), lambda i: (0, i, 0)),
            pl.BlockSpec((F, BS, D), lambda i: (0, i, 0)),
            full(dense_w0), full(b2(dense_b0)),
            full(dense_w1), full(b2(dense_b1)),
            full(dense_w2), full(b2(dense_b2)),
            full(wd), full(w3),
            full(b2(over_b0)), full(over_w1), full(b2(over_b1)),
            full(over_w2), full(b2(over_b2)), full(over_w3), full(b2(over_b3)),
        ],
        out_specs=pl.BlockSpec((BS, 1), lambda i: (i, 0)),
        out_shape=jax.ShapeDtypeStruct((B, 1), jnp.float32),
    )(dense_features, sp, parity,
      dense_w0, b2(dense_b0), dense_w1, b2(dense_b1), dense_w2, b2(dense_b2),
      wd, w3, b2(over_b0), over_w1, b2(over_b1), over_w2, b2(over_b2),
      over_w3, b2(over_b3))
    return out


# speedup vs baseline: 1.0505x; 1.0505x over previous
import functools
import numpy as np
import jax
import jax.numpy as jnp
from jax import lax
from jax.experimental import pallas as pl
from jax.experimental.pallas import tpu as pltpu
from jax.experimental.pallas import tpu_sc as plsc

F = 26
B = 4096
D = 64
PER_TABLE = 38462
TOTAL_VOCAB = F * PER_TABLE
NUM_F = F + 1
N = F * B
NW = 32
ROWS_PER_W = N // NW
CHUNK = 128
NCHUNK = ROWS_PER_W // CHUNK
IDX_ROWS = N // CHUNK
BS = 128
_LI, _LJ = np.tril_indices(NUM_F, k=-1)


TW = 16384
HALF = 507904
TGRID = HALF // TW
PV_ROWS = HALF
_NBLK = -(-TOTAL_VOCAB // TW) - 1


def _tc_trans_body(tina, tinb, tout):
    tout[...] = jnp.concatenate(
        [jnp.transpose(tina[...]), jnp.transpose(tinb[...])], axis=1)


def _tc_transpose(tbl_t):
    return pl.pallas_call(
        _tc_trans_body,
        grid=(TGRID,),
        in_specs=[
            pl.BlockSpec((D, TW), lambda i: (0, i)),
            pl.BlockSpec((D, TW), lambda i: (0, jnp.minimum(TGRID + i, _NBLK))),
        ],
        out_specs=pl.BlockSpec((TW, 2 * D), lambda i: (i, 0)),
        out_shape=jax.ShapeDtypeStruct((PV_ROWS, 2 * D), jnp.float32),
    )(tbl_t, tbl_t)


@functools.lru_cache(maxsize=1)
def _make_sc_gather():
    mesh = plsc.VectorSubcoreMesh(core_axis_name="c", subcore_axis_name="s")

    @functools.partial(
        pl.kernel,
        mesh=mesh,
        out_type=jax.ShapeDtypeStruct((F, B, 2 * D), jnp.float32),
        scratch_types=[
            pltpu.VMEM((NCHUNK, CHUNK), jnp.int32),
            pltpu.VMEM((CHUNK, 2 * D), jnp.float32),
            pltpu.VMEM((CHUNK, 2 * D), jnp.float32),
            pltpu.SemaphoreType.DMA,
            pltpu.SemaphoreType.DMA,
        ],
    )
    def _sc_gather(idx_hbm, table_hbm, out_hbm, idx_v, rows_a, rows_b, s_a, s_b):
        wid = lax.axis_index("s") * 2 + lax.axis_index("c")
        pltpu.sync_copy(idx_hbm.at[wid], idx_v)
        bufs = (rows_a, rows_b)
        sems = (s_a, s_b)
        pltpu.async_copy(table_hbm.at[idx_v.at[0]], rows_a, s_a)
        for g in range(NCHUNK):
            if g + 1 < NCHUNK:
                pltpu.async_copy(table_hbm.at[idx_v.at[g + 1]],
                                 bufs[(g + 1) % 2], sems[(g + 1) % 2])
            pltpu.make_async_copy(table_hbm.at[idx_v.at[g]],
                                  bufs[g % 2], sems[g % 2]).wait()
            gidx = wid * NCHUNK + g
            f = gidx // (B // CHUNK)
            col = (gidx % (B // CHUNK)) * CHUNK
            pltpu.sync_copy(bufs[g % 2], out_hbm.at[f, pl.ds(col, CHUNK)])

    return _sc_gather


def _tc_body(df, sp, par, dw0, db0, dw1, db1, dw2, db2,
             wd, w3, ob0, ow1, ob1, ow2, ob2, ow3, ob3, out):
    f32 = jnp.float32
    x = jnp.maximum(jnp.dot(df[...], dw0[...], preferred_element_type=f32) + db0[...], 0.0)
    x = jnp.maximum(jnp.dot(x, dw1[...], preferred_element_type=f32) + db1[...], 0.0)
    dense_emb = jnp.maximum(jnp.dot(x, dw2[...], preferred_element_type=f32) + db2[...], 0.0)

    pairs = sp[...]
    csp = jnp.where(par[...] != 0, pairs[:, :, D:], pairs[:, :, :D])
    c = jnp.concatenate([dense_emb[None], csp], axis=0)
    cb = c.astype(jnp.bfloat16)
    z = lax.dot_general(cb, cb, (((2,), (2,)), ((1,), (1,))),
                        preferred_element_type=f32)

    y = jnp.dot(dense_emb, wd[...], preferred_element_type=f32) + ob0[...]
    for f in range(NUM_F):
        y = y + jnp.dot(z[:, f, :], w3[f], preferred_element_type=f32)
    y = jnp.maximum(y, 0.0)
    y = jnp.maximum(jnp.dot(y, ow1[...], preferred_element_type=f32) + ob1[...], 0.0)
    y = jnp.maximum(jnp.dot(y, ow2[...], preferred_element_type=f32) + ob2[...], 0.0)
    out[...] = jnp.dot(y, ow3[...], preferred_element_type=f32) + ob3[...]


def kernel(dense_features, sparse_values, sparse_offsets, emb_table,
           dense_w0, dense_b0, dense_w1, dense_b1, dense_w2, dense_b2,
           over_w0, over_b0, over_w1, over_b1, over_w2, over_b2,
           over_w3, over_b3):
    del sparse_offsets
    half_flag = sparse_values >= HALF
    pair_idx = jnp.where(half_flag, sparse_values - HALF,
                         sparse_values).reshape(NW, NCHUNK, CHUNK)
    parity = jnp.broadcast_to(
        half_flag.astype(jnp.int8).reshape(F, B)[:, :, None], (F, B, D))
    pair_view = _tc_transpose(emb_table.T)
    sp = _make_sc_gather()(pair_idx, pair_view)

    wd = over_w0[:D]
    w3 = jnp.zeros((NUM_F, NUM_F, over_w0.shape[1]), jnp.float32)
    w3 = w3.at[_LI, _LJ, :].set(over_w0[D:])

    grid = B // BS
    full = lambda a: pl.BlockSpec(a.shape, lambda i: (0,) * a.ndim)
    b2 = lambda b: b.reshape(1, -1)

    out = pl.pallas_call(
        _tc_body,
        grid=(grid,),
        in_specs=[
            pl.BlockSpec((BS, 13), lambda i: (i, 0)),
            pl.BlockSpec((F, BS, 2 * D), lambda i: (0, i, 0)),
            pl.BlockSpec((F, BS, D), lambda i: (0, i, 0)),
            full(dense_w0), full(b2(dense_b0)),
            full(dense_w1), full(b2(dense_b1)),
            full(dense_w2), full(b2(dense_b2)),
            full(wd), full(w3),
            full(b2(over_b0)), full(over_w1), full(b2(over_b1)),
            full(over_w2), full(b2(over_b2)), full(over_w3), full(b2(over_b3)),
        ],
        out_specs=pl.BlockSpec((BS, 1), lambda i: (i, 0)),
        out_shape=jax.ShapeDtypeStruct((B, 1), jnp.float32),
    )(dense_features, sp, parity,
      dense_w0, b2(dense_b0), dense_w1, b2(dense_b1), dense_w2, b2(dense_b2),
      wd, w3, b2(over_b0), over_w1, b2(over_b1), over_w2, b2(over_b2),
      over_w3, b2(over_b3))
    return out


# speedup vs baseline: 1.1087x; 1.0554x over previous
import functools
import numpy as np
import jax
import jax.numpy as jnp
from jax import lax
from jax.experimental import pallas as pl
from jax.experimental.pallas import tpu as pltpu
from jax.experimental.pallas import tpu_sc as plsc

F = 26
B = 4096
D = 64
PER_TABLE = 38462
TOTAL_VOCAB = F * PER_TABLE
NUM_F = F + 1
N = F * B
NW = 32
ROWS_PER_W = N // NW
CHUNK = 128
NCHUNK = ROWS_PER_W // CHUNK
IDX_ROWS = N // CHUNK
BS = 256
_LI, _LJ = np.tril_indices(NUM_F, k=-1)


TW = 16384
HALF = 507904
TGRID = HALF // TW
PV_ROWS = HALF
_NBLK = -(-TOTAL_VOCAB // TW) - 1


def _tc_trans_body(tina, tinb, tout):
    tout[...] = jnp.concatenate(
        [jnp.transpose(tina[...]), jnp.transpose(tinb[...])], axis=1)


def _tc_transpose(tbl_t):
    return pl.pallas_call(
        _tc_trans_body,
        grid=(TGRID,),
        in_specs=[
            pl.BlockSpec((D, TW), lambda i: (0, i)),
            pl.BlockSpec((D, TW), lambda i: (0, jnp.minimum(TGRID + i, _NBLK))),
        ],
        out_specs=pl.BlockSpec((TW, 2 * D), lambda i: (i, 0)),
        out_shape=jax.ShapeDtypeStruct((PV_ROWS, 2 * D), jnp.float32),
    )(tbl_t, tbl_t)


@functools.lru_cache(maxsize=1)
def _make_sc_gather():
    mesh = plsc.VectorSubcoreMesh(core_axis_name="c", subcore_axis_name="s")

    @functools.partial(
        pl.kernel,
        mesh=mesh,
        out_type=jax.ShapeDtypeStruct((F, B, 2 * D), jnp.float32),
        scratch_types=[
            pltpu.VMEM((NCHUNK, CHUNK), jnp.int32),
            pltpu.VMEM((CHUNK, 2 * D), jnp.float32),
            pltpu.VMEM((CHUNK, 2 * D), jnp.float32),
            pltpu.SemaphoreType.DMA,
            pltpu.SemaphoreType.DMA,
        ],
    )
    def _sc_gather(idx_hbm, table_hbm, out_hbm, idx_v, rows_a, rows_b, s_a, s_b):
        wid = lax.axis_index("s") * 2 + lax.axis_index("c")
        pltpu.sync_copy(idx_hbm.at[wid], idx_v)
        bufs = (rows_a, rows_b)
        sems = (s_a, s_b)
        pltpu.async_copy(table_hbm.at[idx_v.at[0]], rows_a, s_a)
        for g in range(NCHUNK):
            if g + 1 < NCHUNK:
                pltpu.async_copy(table_hbm.at[idx_v.at[g + 1]],
                                 bufs[(g + 1) % 2], sems[(g + 1) % 2])
            pltpu.make_async_copy(table_hbm.at[idx_v.at[g]],
                                  bufs[g % 2], sems[g % 2]).wait()
            gidx = wid * NCHUNK + g
            f = gidx // (B // CHUNK)
            col = (gidx % (B // CHUNK)) * CHUNK
            pltpu.sync_copy(bufs[g % 2], out_hbm.at[f, pl.ds(col, CHUNK)])

    return _sc_gather


def _tc_body(df, sp, par, dw0, db0, dw1, db1, dw2, db2,
             wd, w3, ob0, ow1, ob1, ow2, ob2, ow3, ob3, out):
    f32 = jnp.float32
    x = jnp.maximum(jnp.dot(df[...], dw0[...], preferred_element_type=f32) + db0[...], 0.0)
    x = jnp.maximum(jnp.dot(x, dw1[...], preferred_element_type=f32) + db1[...], 0.0)
    dense_emb = jnp.maximum(jnp.dot(x, dw2[...], preferred_element_type=f32) + db2[...], 0.0)

    pairs = sp[...]
    csp = jnp.where(par[...] != 0, pairs[:, :, D:], pairs[:, :, :D])
    c = jnp.concatenate([dense_emb[None], csp], axis=0)
    cb = c.astype(jnp.bfloat16)
    z = lax.dot_general(cb, cb, (((2,), (2,)), ((1,), (1,))),
                        preferred_element_type=f32)

    y = jnp.dot(dense_emb, wd[...], preferred_element_type=f32) + ob0[...]
    for f in range(NUM_F):
        y = y + jnp.dot(z[:, f, :], w3[f], preferred_element_type=f32)
    y = jnp.maximum(y, 0.0)
    y = jnp.maximum(jnp.dot(y, ow1[...], preferred_element_type=f32) + ob1[...], 0.0)
    y = jnp.maximum(jnp.dot(y, ow2[...], preferred_element_type=f32) + ob2[...], 0.0)
    out[...] = jnp.dot(y, ow3[...], preferred_element_type=f32) + ob3[...]


def kernel(dense_features, sparse_values, sparse_offsets, emb_table,
           dense_w0, dense_b0, dense_w1, dense_b1, dense_w2, dense_b2,
           over_w0, over_b0, over_w1, over_b1, over_w2, over_b2,
           over_w3, over_b3):
    del sparse_offsets
    half_flag = sparse_values >= HALF
    pair_idx = jnp.where(half_flag, sparse_values - HALF,
                         sparse_values).reshape(NW, NCHUNK, CHUNK)
    parity = jnp.broadcast_to(
        half_flag.astype(jnp.int8).reshape(F, B)[:, :, None], (F, B, D))
    pair_view = _tc_transpose(emb_table.T)
    sp = _make_sc_gather()(pair_idx, pair_view)

    wd = over_w0[:D]
    w3 = jnp.zeros((NUM_F, NUM_F, over_w0.shape[1]), jnp.float32)
    w3 = w3.at[_LI, _LJ, :].set(over_w0[D:])

    grid = B // BS
    full = lambda a: pl.BlockSpec(a.shape, lambda i: (0,) * a.ndim)
    b2 = lambda b: b.reshape(1, -1)

    out = pl.pallas_call(
        _tc_body,
        grid=(grid,),
        in_specs=[
            pl.BlockSpec((BS, 13), lambda i: (i, 0)),
            pl.BlockSpec((F, BS, 2 * D), lambda i: (0, i, 0)),
            pl.BlockSpec((F, BS, D), lambda i: (0, i, 0)),
            full(dense_w0), full(b2(dense_b0)),
            full(dense_w1), full(b2(dense_b1)),
            full(dense_w2), full(b2(dense_b2)),
            full(wd), full(w3),
            full(b2(over_b0)), full(over_w1), full(b2(over_b1)),
            full(over_w2), full(b2(over_b2)), full(over_w3), full(b2(over_b3)),
        ],
        out_specs=pl.BlockSpec((BS, 1), lambda i: (i, 0)),
        out_shape=jax.ShapeDtypeStruct((B, 1), jnp.float32),
    )(dense_features, sp, parity,
      dense_w0, b2(dense_b0), dense_w1, b2(dense_b1), dense_w2, b2(dense_b2),
      wd, w3, b2(over_b0), over_w1, b2(over_b1), over_w2, b2(over_b2),
      over_w3, b2(over_b3))
    return out
